# Initial kernel scaffold; baseline (speedup 1.0000x reference)
import jax
import jax.numpy as jnp
from jax.experimental import pallas as pl


def kernel(flows, boxes):
    raise NotImplementedError("write your pallas kernel here")


# speedup vs baseline: 68.3994x; 68.3994x over previous
import functools

import jax
import jax.numpy as jnp
from jax import lax
from jax.experimental import pallas as pl
from jax.experimental.pallas import tpu as pltpu

N_BINS = 8
OUT = 224
H = W = 512


def _interp_matrix(lo, frac, size):
    cols = lax.broadcasted_iota(jnp.int32, (OUT, size), 1)
    hi = jnp.minimum(lo + 1, size - 1)
    return (jnp.where(cols == lo, 1.0 - frac, 0.0)
            + jnp.where(cols == hi, frac, 0.0))


def _coords(start, extent, size):
    g = (lax.broadcasted_iota(jnp.int32, (OUT, 1), 0).astype(jnp.float32)
         + 0.5) / OUT
    c = jnp.clip(start + g * extent, 0.0, size - 1.0)
    c0 = jnp.floor(c)
    return c0.astype(jnp.int32), c - c0


def _tc_kernel(boxes_ref, flows_ref, out_ref):
    m = pl.program_id(0)
    bidx = boxes_ref[m, 0].astype(jnp.int32)
    x1 = boxes_ref[m, 1]
    y1 = boxes_ref[m, 2]
    roi_w = jnp.maximum(boxes_ref[m, 3] - x1, 1.0)
    roi_h = jnp.maximum(boxes_ref[m, 4] - y1, 1.0)

    y0i, ly = _coords(y1, roi_h, H)
    x0i, lx = _coords(x1, roi_w, W)
    wy = _interp_matrix(y0i, ly, H)
    wx = _interp_matrix(x0i, lx, W)

    def sample(c):
        img = flows_ref[bidx, c]
        tmp = jax.lax.dot_general(wy, img, (((1,), (0,)), ((), ())),
                                  preferred_element_type=jnp.float32)
        return jax.lax.dot_general(tmp, wx, (((1,), (1,)), ((), ())),
                                   preferred_element_type=jnp.float32)

    a = sample(0)
    b = sample(1)
    mag = jnp.sqrt(a * a + b * b)
    theta = jnp.arctan2(a, b)
    bins = jnp.clip(jnp.floor((theta + jnp.pi) / (2.0 * jnp.pi) * N_BINS),
                    0, N_BINS - 1).astype(jnp.int32)

    bin_ids = lax.broadcasted_iota(jnp.int32, (N_BINS, OUT, OUT), 0)
    masks = (bins[None, :, :] == bin_ids)
    hist = jnp.sum(jnp.where(masks, mag[None, :, :], 0.0), axis=(1, 2))
    cnt = jnp.sum(masks.astype(jnp.float32), axis=(1, 2))
    res = jnp.where(cnt != 0, hist / jnp.where(cnt != 0, cnt, 1.0), 0.0)
    out_ref[0, :, :] = res.reshape(1, N_BINS)


def kernel(flows, boxes):
    M = boxes.shape[0]
    out = pl.pallas_call(
        _tc_kernel,
        grid=(M,),
        in_specs=[
            pl.BlockSpec(memory_space=pltpu.SMEM),
            pl.BlockSpec((flows.shape[0], 2, H, W), lambda m: (0, 0, 0, 0)),
        ],
        out_specs=pl.BlockSpec((1, 1, N_BINS), lambda m: (m, 0, 0)),
        out_shape=jax.ShapeDtypeStruct((M, 1, N_BINS), jnp.float32),
    )(boxes, flows)
    return out.reshape(M, N_BINS)
